# baseline (speedup 1.0000x reference)
import functools

import jax
import jax.numpy as jnp
import numpy as np
from jax import lax
from jax.experimental import pallas as pl
from jax.experimental.pallas import tpu as pltpu
from jax.experimental.pallas import tpu_sc as plsc

N = 10000
E = 320000
D_IN = 128
HID = 16

NC = 2
NS = 16
NW = NC * NS
LANES = 16

EB = 128
NBLK = 80
EPW = NBLK * EB
SLOTS = 4
SB = 256
SPB = SB // EB
NSB = EPW // SB
E_PAD = EPW * NW
NROWS = 10240
ROWS_PER_TILE = NROWS // NS
ACC_W = 2 * HID

_GDN = lax.GatherDimensionNumbers(
    offset_dims=(), collapsed_slice_dims=(0,), start_index_map=(0,))


def _shuf(v, perm):
  return lax.gather(v, perm.reshape(LANES, 1), dimension_numbers=_GDN,
                    slice_sizes=(1,),
                    mode=lax.GatherScatterMode.PROMISE_IN_BOUNDS)


def _edge_kernel_body(xors, srcp, dstp, xl, xr, atth, acc_out,
                      attb, sidx, didx,
                      xlb0, xrb0, xlb1, xrb1, xlb2, xrb2, xlb3, xrb3,
                      exb0, msgb0, exb1, msgb1, exb2, msgb2, exb3, msgb3,
                      zbuf, den_sh, msg_sh,
                      semg0, semg1, semg2, semg3,
                      sems0, sems1, sems2, sems3):
  c = lax.axis_index("c")
  s = lax.axis_index("s")
  wid = s * NC + c

  xlbs = (xlb0, xlb1, xlb2, xlb3)
  xrbs = (xrb0, xrb1, xrb2, xrb3)
  exbs = (exb0, exb1, exb2, exb3)
  msgbs = (msgb0, msgb1, msgb2, msgb3)
  semgs = (semg0, semg1, semg2, semg3)
  semss = (sems0, sems1, sems2, sems3)

  zero = jnp.zeros((LANES,), jnp.float32)

  def zb(i, carry):
    zbuf[i, :] = zero
    return carry

  lax.fori_loop(0, ROWS_PER_TILE // 2, zb, 0)
  zbase = s * ROWS_PER_TILE
  half = ROWS_PER_TILE // 2
  pltpu.sync_copy(zbuf, den_sh.at[pl.ds(zbase, half)])
  pltpu.sync_copy(zbuf, den_sh.at[pl.ds(zbase + half, half)])
  pltpu.sync_copy(zbuf, msg_sh.at[pl.ds(zbase, half)])
  pltpu.sync_copy(zbuf, msg_sh.at[pl.ds(zbase + half, half)])
  pltpu.sync_copy(atth, attb)
  pltpu.sync_copy(srcp.at[wid], sidx)
  pltpu.sync_copy(dstp.at[wid], didx)
  plsc.subcore_barrier()

  att = attb[...]
  att5 = att * jnp.float32(0.2)
  perms = [jnp.arange(LANES, dtype=jnp.int32) ^ x for x in xors]

  def gather_start(g, slot):
    for j in range(SPB):
      pltpu.make_async_copy(xl.at[sidx.at[g * SPB + j]],
                            xlbs[slot].at[pl.ds(j * EB, EB)],
                            semgs[slot]).start()
      pltpu.make_async_copy(xr.at[didx.at[g * SPB + j]],
                            xrbs[slot].at[pl.ds(j * EB, EB)],
                            semgs[slot]).start()

  def gather_wait(slot):
    for j in range(SPB):
      pltpu.make_async_copy(xl.at[sidx.at[0]],
                            xlbs[slot].at[pl.ds(j * EB, EB)],
                            semgs[slot]).wait()
      pltpu.make_async_copy(xr.at[didx.at[0]],
                            xrbs[slot].at[pl.ds(j * EB, EB)],
                            semgs[slot]).wait()

  def scatter_start(g, slot):
    for j in range(SPB):
      pltpu.make_async_copy(exbs[slot].at[pl.ds(j * EB, EB)],
                            den_sh.at[didx.at[g * SPB + j]],
                            semss[slot]).start(add=True)
      pltpu.make_async_copy(msgbs[slot].at[pl.ds(j * EB, EB)],
                            msg_sh.at[didx.at[g * SPB + j]],
                            semss[slot]).start(add=True)

  def scatter_wait(slot):
    for j in range(SPB):
      pltpu.make_async_copy(exbs[slot].at[pl.ds(j * EB, EB)],
                            den_sh.at[didx.at[0]],
                            semss[slot]).wait()
      pltpu.make_async_copy(msgbs[slot].at[pl.ds(j * EB, EB)],
                            msg_sh.at[didx.at[0]],
                            semss[slot]).wait()

  def compute(slot):
    xlb, xrb, exb, msgb = xlbs[slot], xrbs[slot], exbs[slot], msgbs[slot]

    @plsc.parallel_loop(0, SB, unroll=4)
    def _edge(i):
      vl = xlb[i, :]
      sv = vl + xrb[i, :]
      p = jnp.where(sv > 0, sv * att, sv * att5)
      for perm in perms:
        p = p + _shuf(p, perm)
      ex = jnp.exp(p)
      exb[i, :] = ex
      msgb[i, :] = ex * vl

  for h in range(SLOTS - 1):
    gather_start(h, h)

  def macro(m, carry):
    for r in range(SLOTS):
      g = m * SLOTS + r
      gather_wait(r)

      @pl.when(m > 0)
      def _():
        scatter_wait(r)

      @pl.when(g + SLOTS - 1 < NSB)
      def _():
        gather_start(g + SLOTS - 1, (r + SLOTS - 1) % SLOTS)

      compute(r)
      scatter_start(g, r)
    return carry

  lax.fori_loop(0, NSB // SLOTS, macro, 0)
  for r in range(SLOTS):
    scatter_wait(r)
  plsc.subcore_barrier()

  pltpu.sync_copy(den_sh.at[pl.ds(zbase, ROWS_PER_TILE)],
                  acc_out.at[c, 0, pl.ds(zbase, ROWS_PER_TILE)])
  pltpu.sync_copy(msg_sh.at[pl.ds(zbase, ROWS_PER_TILE)],
                  acc_out.at[c, 1, pl.ds(zbase, ROWS_PER_TILE)])


def _make_edge_kernel(xors):
  mesh = plsc.VectorSubcoreMesh(core_axis_name="c", subcore_axis_name="s",
                                num_cores=NC, num_subcores=NS)
  return pl.kernel(
      functools.partial(_edge_kernel_body, xors),
      out_type=jax.ShapeDtypeStruct((NC, 2, NROWS, HID), jnp.float32),
      mesh=mesh,
      scratch_types=[
          pltpu.VMEM((LANES,), jnp.float32),
          pltpu.VMEM((NBLK, EB), jnp.int32),
          pltpu.VMEM((NBLK, EB), jnp.int32),
      ] + [pltpu.VMEM((SB, HID), jnp.float32)] * 16 + [
          pltpu.VMEM((ROWS_PER_TILE // 2, HID), jnp.float32),
          pltpu.VMEM_SHARED((NROWS, HID), jnp.float32),
          pltpu.VMEM_SHARED((NROWS, HID), jnp.float32),
      ] + [pltpu.SemaphoreType.DMA] * 8,
      compiler_params=pltpu.CompilerParams(use_tc_tiling_on_sc=False),
      name="gat_edge_pass",
  )


_edge_l1 = _make_edge_kernel((1, 2))
_edge_l2 = _make_edge_kernel((1, 2, 4, 8))

_G1 = np.kron(np.eye(4, dtype=np.float32), np.ones((4, 4), np.float32))
_G2 = np.ones((HID, HID), np.float32)


def _leaky(v):
  return jnp.where(v > 0, v, v * jnp.float32(0.2))


def _proj1_body(x_ref, w_ref, b_ref, ol_ref, or_ref):
  acc = jnp.dot(x_ref[...], w_ref[...],
                preferred_element_type=jnp.float32) + b_ref[...]
  ol_ref[...] = acc[:, :HID]
  or_ref[...] = acc[:, HID:]


def _proj1(xpad, wcat, bcat):
  return pl.pallas_call(
      _proj1_body,
      out_shape=[
          jax.ShapeDtypeStruct((NROWS, HID), jnp.float32),
          jax.ShapeDtypeStruct((NROWS, HID), jnp.float32),
      ],
  )(xpad, wcat, bcat)


def _fuse_body(acc_ref, xl_ref, xr_ref, att_ref, g_ref, b1_ref, w_ref,
               b2_ref, ol_ref, or_ref):
  xl = xl_ref[...]
  lg = jnp.dot(_leaky(xl + xr_ref[...]) * att_ref[...], g_ref[...],
               preferred_element_type=jnp.float32)
  ex = jnp.exp(lg)
  den = acc_ref[0, 0] + acc_ref[1, 0] + ex
  msg = acc_ref[0, 1] + acc_ref[1, 1] + ex * xl
  h = jnp.maximum(msg / (den + 1e-16) + b1_ref[...], 0.0)
  acc = jnp.dot(h, w_ref[...], preferred_element_type=jnp.float32) + b2_ref[...]
  ol_ref[...] = acc[:, :HID]
  or_ref[...] = acc[:, HID:]


def _fuse(acc1, xl1, xr1, att1v, bias1, wcat2, bcat2):
  return pl.pallas_call(
      _fuse_body,
      out_shape=[
          jax.ShapeDtypeStruct((NROWS, HID), jnp.float32),
          jax.ShapeDtypeStruct((NROWS, HID), jnp.float32),
      ],
  )(acc1, xl1, xr1, att1v, _G1, bias1, wcat2, bcat2)


def _final_body(acc_ref, xl_ref, xr_ref, att_ref, g_ref, b_ref, o_ref):
  xl = xl_ref[...]
  lg = jnp.dot(_leaky(xl + xr_ref[...]) * att_ref[...], g_ref[...],
               preferred_element_type=jnp.float32)
  ex = jnp.exp(lg)
  den = acc_ref[0, 0] + acc_ref[1, 0] + ex
  msg = acc_ref[0, 1] + acc_ref[1, 1] + ex * xl
  out = msg / (den + 1e-16) + b_ref[...]
  o_ref[...] = out[:N]


def _final(acc2, xl2, xr2, att2v, bias2):
  return pl.pallas_call(
      _final_body,
      out_shape=jax.ShapeDtypeStruct((N, HID), jnp.float32),
  )(acc2, xl2, xr2, att2v, _G2, bias2)


@jax.jit
def _impl(x, edge_index, Wl1, bl1, Wr1, br1, att1, bias1,
          Wl2, bl2, Wr2, br2, att2, bias2):
  srcp = jnp.pad(edge_index[0], (0, E_PAD - E),
                 constant_values=N).reshape(NW, NBLK, EB)
  dstp = jnp.pad(edge_index[1], (0, E_PAD - E),
                 constant_values=N).reshape(NW, NBLK, EB)

  xpad = jnp.pad(x, ((0, NROWS - N), (0, 0)))
  w1 = jnp.concatenate([Wl1, Wr1], axis=1)
  b1 = jnp.concatenate([bl1, br1]).reshape(1, 2 * HID)
  xl1, xr1 = _proj1(xpad, w1, b1)

  att1v = att1.reshape(1, HID)
  acc1 = _edge_l1(srcp, dstp, xl1, xr1, att1.reshape(HID))

  w2 = jnp.concatenate([Wl2, Wr2], axis=1)
  b2 = jnp.concatenate([bl2, br2]).reshape(1, 2 * HID)
  xl2, xr2 = _fuse(acc1, xl1, xr1, att1v, bias1.reshape(1, HID), w2, b2)

  att2v = att2.reshape(1, HID)
  acc2 = _edge_l2(srcp, dstp, xl2, xr2, att2.reshape(HID))

  return _final(acc2, xl2, xr2, att2v, bias2.reshape(1, HID))


def kernel(x, edge_index, Wl1, bl1, Wr1, br1, att1, bias1,
           Wl2, bl2, Wr2, br2, att2, bias2):
  return _impl(x, edge_index, Wl1, bl1, Wr1, br1, att1, bias1,
               Wl2, bl2, Wr2, br2, att2, bias2)


# speedup vs baseline: 1.4359x; 1.4359x over previous
import functools

import jax
import jax.numpy as jnp
import numpy as np
from jax import lax
from jax.experimental import pallas as pl
from jax.experimental.pallas import tpu as pltpu
from jax.experimental.pallas import tpu_sc as plsc

N = 10000
E = 320000
D_IN = 128
HID = 16

NC = 2
NS = 16
NW = NC * NS
LANES = 16

EB = 128
NBLK = 80
EPW = NBLK * EB
SLOTS = 4
SB = 256
SPB = SB // EB
NSB = EPW // SB
E_PAD = EPW * NW
NROWS = 10240
ROWS_PER_TILE = NROWS // NS
ACC_W = 2 * HID

_GDN = lax.GatherDimensionNumbers(
    offset_dims=(), collapsed_slice_dims=(0,), start_index_map=(0,))


def _shuf(v, perm):
  return lax.gather(v, perm.reshape(LANES, 1), dimension_numbers=_GDN,
                    slice_sizes=(1,),
                    mode=lax.GatherScatterMode.PROMISE_IN_BOUNDS)


def _edge_kernel_body(xors, srcp, dstp, xl, xr, atth, acc_out,
                      attb, sidx, didx,
                      xlb0, xrb0, xlb1, xrb1, xlb2, xrb2, xlb3, xrb3,
                      exb0, msgb0, exb1, msgb1, exb2, msgb2, exb3, msgb3,
                      zbuf, den_sh, msg_sh,
                      semg0, semg1, semg2, semg3,
                      sems0, sems1, sems2, sems3):
  c = lax.axis_index("c")
  s = lax.axis_index("s")
  wid = s * NC + c

  xlbs = (xlb0, xlb1, xlb2, xlb3)
  xrbs = (xrb0, xrb1, xrb2, xrb3)
  exbs = (exb0, exb1, exb2, exb3)
  msgbs = (msgb0, msgb1, msgb2, msgb3)
  semgs = (semg0, semg1, semg2, semg3)
  semss = (sems0, sems1, sems2, sems3)

  zero = jnp.zeros((LANES,), jnp.float32)

  def zb(i, carry):
    zbuf[i, :] = zero
    return carry

  lax.fori_loop(0, ROWS_PER_TILE // 2, zb, 0)
  zbase = s * ROWS_PER_TILE
  half = ROWS_PER_TILE // 2
  pltpu.sync_copy(zbuf, den_sh.at[pl.ds(zbase, half)])
  pltpu.sync_copy(zbuf, den_sh.at[pl.ds(zbase + half, half)])
  pltpu.sync_copy(zbuf, msg_sh.at[pl.ds(zbase, half)])
  pltpu.sync_copy(zbuf, msg_sh.at[pl.ds(zbase + half, half)])
  pltpu.sync_copy(atth, attb)
  pltpu.sync_copy(srcp.at[wid], sidx)
  pltpu.sync_copy(dstp.at[wid], didx)
  plsc.subcore_barrier()

  att = attb[...]
  att5 = att * jnp.float32(0.2)
  perms = [jnp.arange(LANES, dtype=jnp.int32) ^ x for x in xors]

  def gather_start(g, slot):
    for j in range(SPB):
      pltpu.make_async_copy(xl.at[sidx.at[g * SPB + j]],
                            xlbs[slot].at[pl.ds(j * EB, EB)],
                            semgs[slot]).start()
      pltpu.make_async_copy(xr.at[didx.at[g * SPB + j]],
                            xrbs[slot].at[pl.ds(j * EB, EB)],
                            semgs[slot]).start()

  def gather_wait(slot):
    for j in range(SPB):
      pltpu.make_async_copy(xl.at[sidx.at[0]],
                            xlbs[slot].at[pl.ds(j * EB, EB)],
                            semgs[slot]).wait()
      pltpu.make_async_copy(xr.at[didx.at[0]],
                            xrbs[slot].at[pl.ds(j * EB, EB)],
                            semgs[slot]).wait()

  def scatter_start(g, slot):
    for j in range(SPB):
      pltpu.make_async_copy(exbs[slot].at[pl.ds(j * EB, EB)],
                            den_sh.at[didx.at[g * SPB + j]],
                            semss[slot]).start(add=True)
      pltpu.make_async_copy(msgbs[slot].at[pl.ds(j * EB, EB)],
                            msg_sh.at[didx.at[g * SPB + j]],
                            semss[slot]).start(add=True)

  def scatter_wait(slot):
    for j in range(SPB):
      pltpu.make_async_copy(exbs[slot].at[pl.ds(j * EB, EB)],
                            den_sh.at[didx.at[0]],
                            semss[slot]).wait()
      pltpu.make_async_copy(msgbs[slot].at[pl.ds(j * EB, EB)],
                            msg_sh.at[didx.at[0]],
                            semss[slot]).wait()

  def compute(slot):
    xlb, xrb, exb, msgb = xlbs[slot], xrbs[slot], exbs[slot], msgbs[slot]

    @plsc.parallel_loop(0, SB, unroll=4)
    def _edge(i):
      vl = xlb[i, :]
      sv = vl + xrb[i, :]
      p = jnp.where(sv > 0, sv * att, sv * att5)
      for perm in perms:
        p = p + _shuf(p, perm)
      ex = jnp.exp(p)
      exb[i, :] = ex
      msgb[i, :] = ex * vl

  for h in range(SLOTS - 1):
    gather_start(h, h)

  def macro(m, carry):
    for r in range(SLOTS):
      g = m * SLOTS + r
      gather_wait(r)

      @pl.when(m > 0)
      def _():
        scatter_wait(r)

      @pl.when(g + SLOTS - 1 < NSB)
      def _():
        gather_start(g + SLOTS - 1, (r + SLOTS - 1) % SLOTS)

      compute(r)
      scatter_start(g, r)
    return carry

  lax.fori_loop(0, NSB // SLOTS, macro, 0)
  for r in range(SLOTS):
    scatter_wait(r)
  plsc.subcore_barrier()

  pltpu.sync_copy(den_sh.at[pl.ds(zbase, ROWS_PER_TILE)],
                  acc_out.at[c, 0, pl.ds(zbase, ROWS_PER_TILE)])
  pltpu.sync_copy(msg_sh.at[pl.ds(zbase, ROWS_PER_TILE)],
                  acc_out.at[c, 1, pl.ds(zbase, ROWS_PER_TILE)])


def _make_edge_kernel(xors):
  mesh = plsc.VectorSubcoreMesh(core_axis_name="c", subcore_axis_name="s",
                                num_cores=NC, num_subcores=NS)
  return pl.kernel(
      functools.partial(_edge_kernel_body, xors),
      out_type=jax.ShapeDtypeStruct((NC, 2, NROWS, HID), jnp.float32),
      mesh=mesh,
      scratch_types=[
          pltpu.VMEM((LANES,), jnp.float32),
          pltpu.VMEM((NBLK, EB), jnp.int32),
          pltpu.VMEM((NBLK, EB), jnp.int32),
      ] + [pltpu.VMEM((SB, HID), jnp.float32)] * 16 + [
          pltpu.VMEM((ROWS_PER_TILE // 2, HID), jnp.float32),
          pltpu.VMEM_SHARED((NROWS, HID), jnp.float32),
          pltpu.VMEM_SHARED((NROWS, HID), jnp.float32),
      ] + [pltpu.SemaphoreType.DMA] * 8,
      compiler_params=pltpu.CompilerParams(use_tc_tiling_on_sc=False),
      name="gat_edge_pass",
  )


_edge_l1 = _make_edge_kernel((1, 2))
_edge_l2 = _make_edge_kernel((1, 2, 4, 8))

_G1 = np.kron(np.eye(4, dtype=np.float32), np.ones((4, 4), np.float32))
_G2 = np.ones((HID, HID), np.float32)


def _leaky(v):
  return jnp.where(v > 0, v, v * jnp.float32(0.2))


def _proj1_body(x_ref, w_ref, b_ref, ol_ref, or_ref):
  acc = jnp.dot(x_ref[...], w_ref[...],
                preferred_element_type=jnp.float32) + b_ref[...]
  ol_ref[...] = acc[:, :HID]
  or_ref[...] = acc[:, HID:]


def _proj1(xpad, wcat, bcat):
  return pl.pallas_call(
      _proj1_body,
      out_shape=[
          jax.ShapeDtypeStruct((NROWS, HID), jnp.float32),
          jax.ShapeDtypeStruct((NROWS, HID), jnp.float32),
      ],
  )(xpad, wcat, bcat)


def _fuse_body(acc_ref, xl_ref, xr_ref, att_ref, g_ref, b1_ref, w_ref,
               b2_ref, ol_ref, or_ref):
  xl = xl_ref[...]
  lg = jnp.dot(_leaky(xl + xr_ref[...]) * att_ref[...], g_ref[...],
               preferred_element_type=jnp.float32)
  ex = jnp.exp(lg)
  den = acc_ref[0, 0] + acc_ref[1, 0] + ex
  msg = acc_ref[0, 1] + acc_ref[1, 1] + ex * xl
  h = jnp.maximum(msg / (den + 1e-16) + b1_ref[...], 0.0)
  acc = jnp.dot(h, w_ref[...], preferred_element_type=jnp.float32) + b2_ref[...]
  ol_ref[...] = acc[:, :HID]
  or_ref[...] = acc[:, HID:]


def _fuse(acc1, xl1, xr1, att1v, bias1, wcat2, bcat2):
  return pl.pallas_call(
      _fuse_body,
      out_shape=[
          jax.ShapeDtypeStruct((NROWS, HID), jnp.float32),
          jax.ShapeDtypeStruct((NROWS, HID), jnp.float32),
      ],
  )(acc1, xl1, xr1, att1v, _G1, bias1, wcat2, bcat2)


def _final_body(acc_ref, xl_ref, xr_ref, att_ref, g_ref, b_ref, o_ref):
  xl = xl_ref[...]
  lg = jnp.dot(_leaky(xl + xr_ref[...]) * att_ref[...], g_ref[...],
               preferred_element_type=jnp.float32)
  ex = jnp.exp(lg)
  den = acc_ref[0, 0] + acc_ref[1, 0] + ex
  msg = acc_ref[0, 1] + acc_ref[1, 1] + ex * xl
  out = msg / (den + 1e-16) + b_ref[...]
  o_ref[...] = out[:N]


def _final(acc2, xl2, xr2, att2v, bias2):
  return pl.pallas_call(
      _final_body,
      out_shape=jax.ShapeDtypeStruct((N, HID), jnp.float32),
  )(acc2, xl2, xr2, att2v, _G2, bias2)


@jax.jit
def _impl(x, edge_index, Wl1, bl1, Wr1, br1, att1, bias1,
          Wl2, bl2, Wr2, br2, att2, bias2):
  padi = np.asarray(N + (np.arange(E_PAD - E) % (NROWS - N)), np.int32)
  srcp = jnp.concatenate([edge_index[0], padi]).reshape(NW, NBLK, EB)
  dstp = jnp.concatenate([edge_index[1], padi]).reshape(NW, NBLK, EB)

  xpad = jnp.pad(x, ((0, NROWS - N), (0, 0)))
  w1 = jnp.concatenate([Wl1, Wr1], axis=1)
  b1 = jnp.concatenate([bl1, br1]).reshape(1, 2 * HID)
  xl1, xr1 = _proj1(xpad, w1, b1)

  att1v = att1.reshape(1, HID)
  acc1 = _edge_l1(srcp, dstp, xl1, xr1, att1.reshape(HID))

  w2 = jnp.concatenate([Wl2, Wr2], axis=1)
  b2 = jnp.concatenate([bl2, br2]).reshape(1, 2 * HID)
  xl2, xr2 = _fuse(acc1, xl1, xr1, att1v, bias1.reshape(1, HID), w2, b2)

  att2v = att2.reshape(1, HID)
  acc2 = _edge_l2(srcp, dstp, xl2, xr2, att2.reshape(HID))

  return _final(acc2, xl2, xr2, att2v, bias2.reshape(1, HID))


def kernel(x, edge_index, Wl1, bl1, Wr1, br1, att1, bias1,
           Wl2, bl2, Wr2, br2, att2, bias2):
  return _impl(x, edge_index, Wl1, bl1, Wr1, br1, att1, bias1,
               Wl2, bl2, Wr2, br2, att2, bias2)


# speedup vs baseline: 1.5489x; 1.0787x over previous
import functools

import jax
import jax.numpy as jnp
import numpy as np
from jax import lax
from jax.experimental import pallas as pl
from jax.experimental.pallas import tpu as pltpu
from jax.experimental.pallas import tpu_sc as plsc

N = 10000
E = 320000
D_IN = 128
HID = 16

NC = 2
NS = 16
NW = NC * NS
LANES = 16

EB = 128
NBLK = 80
EPW = NBLK * EB
SLOTS = 4
SB = 256
SPB = SB // EB
NSB = EPW // SB
E_PAD = EPW * NW
NROWS = 10240
ROWS_PER_TILE = NROWS // NS
ACC_W = 2 * HID

_GDN = lax.GatherDimensionNumbers(
    offset_dims=(), collapsed_slice_dims=(0,), start_index_map=(0,))


def _shuf(v, perm):
  return lax.gather(v, perm.reshape(LANES, 1), dimension_numbers=_GDN,
                    slice_sizes=(1,),
                    mode=lax.GatherScatterMode.PROMISE_IN_BOUNDS)


def _edge_kernel_body(xors, srcp, dstp, xl, xr, atth, acc_out,
                      attb, sidx, didx,
                      xlb0, xrb0, xlb1, xrb1, xlb2, xrb2, xlb3, xrb3,
                      exb0, msgb0, exb1, msgb1, exb2, msgb2, exb3, msgb3,
                      zbuf, den_sh, msg_sh,
                      semg0, semg1, semg2, semg3,
                      sems0, sems1, sems2, sems3):
  c = lax.axis_index("c")
  s = lax.axis_index("s")
  wid = s * NC + c

  xlbs = (xlb0, xlb1, xlb2, xlb3)
  xrbs = (xrb0, xrb1, xrb2, xrb3)
  exbs = (exb0, exb1, exb2, exb3)
  msgbs = (msgb0, msgb1, msgb2, msgb3)
  semgs = (semg0, semg1, semg2, semg3)
  semss = (sems0, sems1, sems2, sems3)

  zero = jnp.zeros((LANES,), jnp.float32)

  def zb(i, carry):
    zbuf[i, :] = zero
    return carry

  lax.fori_loop(0, ROWS_PER_TILE // 2, zb, 0)
  zbase = s * ROWS_PER_TILE
  half = ROWS_PER_TILE // 2
  pltpu.sync_copy(zbuf, den_sh.at[pl.ds(zbase, half)])
  pltpu.sync_copy(zbuf, den_sh.at[pl.ds(zbase + half, half)])
  pltpu.sync_copy(zbuf, msg_sh.at[pl.ds(zbase, half)])
  pltpu.sync_copy(zbuf, msg_sh.at[pl.ds(zbase + half, half)])
  pltpu.sync_copy(atth, attb)
  pltpu.sync_copy(srcp.at[pl.ds(wid * NBLK, NBLK)], sidx)
  pltpu.sync_copy(dstp.at[pl.ds(wid * NBLK, NBLK)], didx)
  plsc.subcore_barrier()

  att = attb[...]
  att5 = att * jnp.float32(0.2)
  perms = [jnp.arange(LANES, dtype=jnp.int32) ^ x for x in xors]

  def gather_start(g, slot):
    for j in range(SPB):
      pltpu.make_async_copy(xl.at[sidx.at[g * SPB + j]],
                            xlbs[slot].at[pl.ds(j * EB, EB)],
                            semgs[slot]).start()
      pltpu.make_async_copy(xr.at[didx.at[g * SPB + j]],
                            xrbs[slot].at[pl.ds(j * EB, EB)],
                            semgs[slot]).start()

  def gather_wait(slot):
    for j in range(SPB):
      pltpu.make_async_copy(xl.at[sidx.at[0]],
                            xlbs[slot].at[pl.ds(j * EB, EB)],
                            semgs[slot]).wait()
      pltpu.make_async_copy(xr.at[didx.at[0]],
                            xrbs[slot].at[pl.ds(j * EB, EB)],
                            semgs[slot]).wait()

  def scatter_start(g, slot):
    for j in range(SPB):
      pltpu.make_async_copy(exbs[slot].at[pl.ds(j * EB, EB)],
                            den_sh.at[didx.at[g * SPB + j]],
                            semss[slot]).start(add=True)
      pltpu.make_async_copy(msgbs[slot].at[pl.ds(j * EB, EB)],
                            msg_sh.at[didx.at[g * SPB + j]],
                            semss[slot]).start(add=True)

  def scatter_wait(slot):
    for j in range(SPB):
      pltpu.make_async_copy(exbs[slot].at[pl.ds(j * EB, EB)],
                            den_sh.at[didx.at[0]],
                            semss[slot]).wait()
      pltpu.make_async_copy(msgbs[slot].at[pl.ds(j * EB, EB)],
                            msg_sh.at[didx.at[0]],
                            semss[slot]).wait()

  def compute(slot):
    xlb, xrb, exb, msgb = xlbs[slot], xrbs[slot], exbs[slot], msgbs[slot]

    @plsc.parallel_loop(0, SB, unroll=4)
    def _edge(i):
      vl = xlb[i, :]
      sv = vl + xrb[i, :]
      p = jnp.where(sv > 0, sv * att, sv * att5)
      for perm in perms:
        p = p + _shuf(p, perm)
      ex = jnp.exp(p)
      exb[i, :] = ex
      msgb[i, :] = ex * vl

  for h in range(SLOTS - 1):
    gather_start(h, h)

  def macro(m, carry):
    for r in range(SLOTS):
      g = m * SLOTS + r
      gather_wait(r)

      @pl.when(m > 0)
      def _():
        scatter_wait(r)

      @pl.when(g + SLOTS - 1 < NSB)
      def _():
        gather_start(g + SLOTS - 1, (r + SLOTS - 1) % SLOTS)

      compute(r)
      scatter_start(g, r)
    return carry

  lax.fori_loop(0, NSB // SLOTS, macro, 0)
  for r in range(SLOTS):
    scatter_wait(r)
  plsc.subcore_barrier()

  pltpu.sync_copy(den_sh.at[pl.ds(zbase, ROWS_PER_TILE)],
                  acc_out.at[c, 0, pl.ds(zbase, ROWS_PER_TILE)])
  pltpu.sync_copy(msg_sh.at[pl.ds(zbase, ROWS_PER_TILE)],
                  acc_out.at[c, 1, pl.ds(zbase, ROWS_PER_TILE)])


def _make_edge_kernel(xors):
  mesh = plsc.VectorSubcoreMesh(core_axis_name="c", subcore_axis_name="s",
                                num_cores=NC, num_subcores=NS)
  return pl.kernel(
      functools.partial(_edge_kernel_body, xors),
      out_type=jax.ShapeDtypeStruct((NC, 2, NROWS, HID), jnp.float32),
      mesh=mesh,
      scratch_types=[
          pltpu.VMEM((LANES,), jnp.float32),
          pltpu.VMEM((NBLK, EB), jnp.int32),
          pltpu.VMEM((NBLK, EB), jnp.int32),
      ] + [pltpu.VMEM((SB, HID), jnp.float32)] * 16 + [
          pltpu.VMEM((ROWS_PER_TILE // 2, HID), jnp.float32),
          pltpu.VMEM_SHARED((NROWS, HID), jnp.float32),
          pltpu.VMEM_SHARED((NROWS, HID), jnp.float32),
      ] + [pltpu.SemaphoreType.DMA] * 8,
      compiler_params=pltpu.CompilerParams(use_tc_tiling_on_sc=False),
      name="gat_edge_pass",
  )


_edge_l1 = _make_edge_kernel((1, 2))
_edge_l2 = _make_edge_kernel((1, 2, 4, 8))

_G1 = np.kron(np.eye(4, dtype=np.float32), np.ones((4, 4), np.float32))
_G2 = np.ones((HID, HID), np.float32)


def _leaky(v):
  return jnp.where(v > 0, v, v * jnp.float32(0.2))


def _proj1_body(x_ref, w_ref, b_ref, ei_ref, padi_ref, ol_ref, or_ref,
                sp_ref, dp_ref):
  acc = jnp.dot(x_ref[...], w_ref[...],
                preferred_element_type=jnp.float32) + b_ref[...]
  ol_ref[...] = acc[:, :HID]
  or_ref[...] = acc[:, HID:]
  padi = padi_ref[...]
  sp_ref[...] = jnp.concatenate([ei_ref[0].reshape(E // EB, EB), padi], axis=0)
  dp_ref[...] = jnp.concatenate([ei_ref[1].reshape(E // EB, EB), padi], axis=0)


def _proj1(xpad, wcat, bcat, edge_index, padi):
  return pl.pallas_call(
      _proj1_body,
      out_shape=[
          jax.ShapeDtypeStruct((NROWS, HID), jnp.float32),
          jax.ShapeDtypeStruct((NROWS, HID), jnp.float32),
          jax.ShapeDtypeStruct((E_PAD // EB, EB), jnp.int32),
          jax.ShapeDtypeStruct((E_PAD // EB, EB), jnp.int32),
      ],
  )(xpad, wcat, bcat, edge_index, padi)


def _fuse_body(acc_ref, xl_ref, xr_ref, att_ref, g_ref, b1_ref, w_ref,
               b2_ref, ol_ref, or_ref):
  xl = xl_ref[...]
  lg = jnp.dot(_leaky(xl + xr_ref[...]) * att_ref[...], g_ref[...],
               preferred_element_type=jnp.float32)
  ex = jnp.exp(lg)
  den = acc_ref[0, 0] + acc_ref[1, 0] + ex
  msg = acc_ref[0, 1] + acc_ref[1, 1] + ex * xl
  h = jnp.maximum(msg / (den + 1e-16) + b1_ref[...], 0.0)
  acc = jnp.dot(h, w_ref[...], preferred_element_type=jnp.float32) + b2_ref[...]
  ol_ref[...] = acc[:, :HID]
  or_ref[...] = acc[:, HID:]


def _fuse(acc1, xl1, xr1, att1v, bias1, wcat2, bcat2):
  return pl.pallas_call(
      _fuse_body,
      out_shape=[
          jax.ShapeDtypeStruct((NROWS, HID), jnp.float32),
          jax.ShapeDtypeStruct((NROWS, HID), jnp.float32),
      ],
  )(acc1, xl1, xr1, att1v, _G1, bias1, wcat2, bcat2)


def _final_body(acc_ref, xl_ref, xr_ref, att_ref, g_ref, b_ref, o_ref):
  xl = xl_ref[...]
  lg = jnp.dot(_leaky(xl + xr_ref[...]) * att_ref[...], g_ref[...],
               preferred_element_type=jnp.float32)
  ex = jnp.exp(lg)
  den = acc_ref[0, 0] + acc_ref[1, 0] + ex
  msg = acc_ref[0, 1] + acc_ref[1, 1] + ex * xl
  out = msg / (den + 1e-16) + b_ref[...]
  o_ref[...] = out[:N]


def _final(acc2, xl2, xr2, att2v, bias2):
  return pl.pallas_call(
      _final_body,
      out_shape=jax.ShapeDtypeStruct((N, HID), jnp.float32),
  )(acc2, xl2, xr2, att2v, _G2, bias2)


@jax.jit
def _impl(x, edge_index, Wl1, bl1, Wr1, br1, att1, bias1,
          Wl2, bl2, Wr2, br2, att2, bias2):
  padi = np.asarray(N + (np.arange(E_PAD - E) % (NROWS - N)),
                    np.int32).reshape((E_PAD - E) // EB, EB)

  xpad = jnp.pad(x, ((0, NROWS - N), (0, 0)))
  w1 = jnp.concatenate([Wl1, Wr1], axis=1)
  b1 = jnp.concatenate([bl1, br1]).reshape(1, 2 * HID)
  xl1, xr1, srcp, dstp = _proj1(xpad, w1, b1, edge_index, padi)

  att1v = att1.reshape(1, HID)
  acc1 = _edge_l1(srcp, dstp, xl1, xr1, att1.reshape(HID))

  w2 = jnp.concatenate([Wl2, Wr2], axis=1)
  b2 = jnp.concatenate([bl2, br2]).reshape(1, 2 * HID)
  xl2, xr2 = _fuse(acc1, xl1, xr1, att1v, bias1.reshape(1, HID), w2, b2)

  att2v = att2.reshape(1, HID)
  acc2 = _edge_l2(srcp, dstp, xl2, xr2, att2.reshape(HID))

  return _final(acc2, xl2, xr2, att2v, bias2.reshape(1, HID))


def kernel(x, edge_index, Wl1, bl1, Wr1, br1, att1, bias1,
           Wl2, bl2, Wr2, br2, att2, bias2):
  return _impl(x, edge_index, Wl1, bl1, Wr1, br1, att1, bias1,
               Wl2, bl2, Wr2, br2, att2, bias2)
